# baseline (speedup 1.0000x reference)
import functools

import jax
import jax.numpy as jnp
from jax import lax
from jax.experimental import pallas as pl
from jax.experimental.pallas import tpu as pltpu
from jax.experimental.pallas import tpu_sc as plsc

N = 10000
E = 320000
D = 128
T = 4

NC = 2
NS = 16
LN = 16

EPT = E // NS
K = 80
S = 2000
NSPC = S // K
NSTRIP = EPT // S
GPR = K // LN
TPC = T // NC
NP = 10240
RPT = NP // NS


def _pre_body(nfT_ref, evT_ref, wcat_ref, b_ref, am_ref, a2_ref):
    x = nfT_ref[0]
    a = jnp.dot(x, wcat_ref[...], preferred_element_type=jnp.float32)
    ev = evT_ref[0, 0]
    a1 = a[:, 0] + b_ref[0, 0]
    am_ref[0, 0] = jnp.where(ev == 1, a1, jnp.float32(-1e9))
    a2_ref[0, 0] = a[:, 1]


def _post_body(nfT_ref, num_ref, den_ref, wn_ref, bn_ref, out_ref):
    x = nfT_ref[0]
    den = jnp.maximum(den_ref[0, 0], jnp.float32(1e-9))
    agg = num_ref[0] / den[:, None]
    h = (jnp.dot(x, wn_ref[:D, :], preferred_element_type=jnp.float32)
         + jnp.dot(agg, wn_ref[D:, :], preferred_element_type=jnp.float32)
         + bn_ref[0])
    out_ref[0] = jnp.maximum(h, 0.0)


def _sc_body(nf_flat, am_hbm, a2_hbm, src_hbm, dst_hbm,
             numer_out, denom_out,
             numer_sh, denom_sh, am_sh, a2_sh,
             src_v, dst_v, amg_v, a2g_v, w_v, csrc_v, cdst_v, cw_v,
             rows_v, zbd_v, gsem):
    cid = lax.axis_index("c")
    sid = lax.axis_index("s")

    zv = jnp.zeros((LN,), jnp.float32)
    zi = jnp.zeros((LN,), jnp.int32)

    def _init_c(g, _):
        sl = pl.ds(g * LN, LN)
        cdst_v[sl] = zi
        return 0
    lax.fori_loop(0, S // LN, _init_c, 0)

    def _init_cs(g, _):
        csrc_v[pl.ds(g * LN, LN)] = zi
        return 0
    lax.fori_loop(0, (S + K) // LN, _init_cs, 0)

    def _zero_zbd(j, _):
        zbd_v[pl.ds(j * LN, LN)] = zv
        return 0
    lax.fori_loop(0, RPT // LN, _zero_zbd, 0)

    for i in range(TPC):
        t = cid * TPC + i
        tbase = t * N

        def _zero_rows(j, _):
            for k in range(D // LN):
                rows_v[0, j, k * LN:(k + 1) * LN] = zv
            return 0
        lax.fori_loop(0, K, _zero_rows, 0)

        for r in range(RPT // K):
            pltpu.sync_copy(rows_v.at[0],
                            numer_sh.at[pl.ds(sid * RPT + r * K, K)])
        pltpu.sync_copy(zbd_v, denom_sh.at[pl.ds(sid * RPT, RPT)])

        @pl.when(sid == 0)
        def _():
            pltpu.sync_copy(am_hbm.at[pl.ds(t * N, N)], am_sh)

        @pl.when(sid == 1)
        def _():
            pltpu.sync_copy(a2_hbm.at[pl.ds(t * N, N)], a2_sh)

        plsc.subcore_barrier()

        def _strip(st, _):
            row = sid * NSTRIP + st
            pltpu.sync_copy(src_hbm.at[row], src_v)
            pltpu.sync_copy(dst_hbm.at[row], dst_v)

            pltpu.async_copy(am_sh.at[src_v], amg_v, gsem).wait()
            pltpu.async_copy(a2_sh.at[dst_v], a2g_v, gsem).wait()

            def _wcalc(g, _):
                sl = pl.ds(g * LN, LN)
                w_v[sl] = jnp.exp(amg_v[sl] + a2g_v[sl])
                src_v[sl] = src_v[sl] + tbase
                return 0
            lax.fori_loop(0, S // LN, _wcalc, 0)

            pltpu.sync_copy(w_v, denom_sh.at[dst_v], add=True)

            def _zcw(g, _):
                cw_v[pl.ds(g * LN, LN)] = zv
                return 0
            lax.fori_loop(0, S // LN, _zcw, 0)

            def _compact(g, off):
                sl = pl.ds(g * LN, LN)
                w16 = w_v[sl]
                msk = w16 > 0.0
                osl = pl.ds(off, LN)
                plsc.store_compressed(cw_v.at[osl], w16, mask=msk)
                plsc.store_compressed(csrc_v.at[osl], src_v[sl], mask=msk)
                plsc.store_compressed(cdst_v.at[osl], dst_v[sl], mask=msk)
                return off + plsc.all_reduce_population_count(msk)[0]
            m = lax.fori_loop(0, S // LN, _compact, jnp.int32(0))
            nchunks = lax.div(m + (K - 1), K)

            def _do_scale(k, buf):
                def _scale(g, _):
                    wg = cw_v[pl.ds(k * K + g * LN, LN)]
                    for l in range(LN):
                        wj = jnp.full((LN,), wg[l])
                        j = g * LN + l
                        for c8 in range(D // LN):
                            sl = pl.ds(c8 * LN, LN)
                            rows_v[buf, j, sl] = rows_v[buf, j, sl] * wj
                    return 0
                lax.fori_loop(0, GPR, _scale, 0)

            def _gather(k, buf):
                return pltpu.async_copy(
                    nf_flat.at[csrc_v.at[pl.ds(k * K, K)]], rows_v.at[buf],
                    gsem)

            def _gwait(k, buf):
                pltpu.make_async_copy(
                    nf_flat.at[csrc_v.at[pl.ds(k * K, K)]], rows_v.at[buf],
                    gsem).wait()

            def _scatter(k, buf):
                pltpu.sync_copy(rows_v.at[buf],
                                numer_sh.at[cdst_v.at[pl.ds(k * K, K)]],
                                add=True)

            _gather(0, 0)

            def _pair(p, _):
                k0 = 2 * p
                _gwait(k0, 0)
                _gather(k0 + 1, 1)
                _do_scale(k0, 0)
                _scatter(k0, 0)
                _gwait(k0 + 1, 1)
                _gather(k0 + 2, 0)
                _do_scale(k0 + 1, 1)
                _scatter(k0 + 1, 1)
                return 0
            lax.fori_loop(0, lax.div(nchunks, 2), _pair, 0)

            @pl.when(lax.rem(nchunks, 2) == 1)
            def _():
                kt = nchunks - 1
                _gwait(kt, 0)
                _gather(kt + 1, 1)
                _do_scale(kt, 0)
                _scatter(kt, 0)

            pltpu.make_async_copy(
                nf_flat.at[csrc_v.at[pl.ds(0, K)]], rows_v.at[0],
                gsem).wait()
            return 0
        lax.fori_loop(0, NSTRIP, _strip, 0)

        plsc.subcore_barrier()

        pbase = t * NP
        pltpu.sync_copy(numer_sh.at[pl.ds(sid * RPT, RPT)],
                        numer_out.at[pl.ds(pbase + sid * RPT, RPT)])
        pltpu.sync_copy(denom_sh.at[pl.ds(sid * RPT, RPT)],
                        denom_out.at[pl.ds(pbase + sid * RPT, RPT)])

        plsc.subcore_barrier()


@functools.cache
def _get_sc_kernel():
    @functools.partial(
        pl.kernel,
        out_type=(jax.ShapeDtypeStruct((T * NP, D), jnp.float32),
                  jax.ShapeDtypeStruct((T * NP,), jnp.float32)),
        mesh=plsc.VectorSubcoreMesh(core_axis_name="c", subcore_axis_name="s",
                                    num_cores=NC, num_subcores=NS),
        scratch_types=(
            pltpu.VMEM_SHARED((NP, D), jnp.float32),
            pltpu.VMEM_SHARED((NP,), jnp.float32),
            pltpu.VMEM_SHARED((N,), jnp.float32),
            pltpu.VMEM_SHARED((N,), jnp.float32),
            pltpu.VMEM((S,), jnp.int32),
            pltpu.VMEM((S,), jnp.int32),
            pltpu.VMEM((S,), jnp.float32),
            pltpu.VMEM((S,), jnp.float32),
            pltpu.VMEM((S,), jnp.float32),
            pltpu.VMEM((S + K,), jnp.int32),
            pltpu.VMEM((S,), jnp.int32),
            pltpu.VMEM((S,), jnp.float32),
            pltpu.VMEM((2, K, D), jnp.float32),
            pltpu.VMEM((RPT,), jnp.float32),
            pltpu.SemaphoreType.DMA,
        ),
        compiler_params=pltpu.CompilerParams(needs_layout_passes=False,
                                             use_tc_tiling_on_sc=False),
    )
    def _sc_kernel(nf_flat, am, a2, srcr, dstr, numer_out, denom_out, *scratch):
        _sc_body(nf_flat, am, a2, srcr, dstr, numer_out, denom_out, *scratch)

    return _sc_kernel


def kernel(nf, edge_index, input_events, W_attn, b_attn, W_node, b_node):
    nfT = jnp.transpose(nf, (1, 0, 2))
    nf_flat = nfT.reshape(T * N, D)
    evT = jnp.transpose(input_events, (1, 0)).reshape(T, 1, N)
    wcat = jnp.concatenate([W_attn[:D], W_attn[D:]], axis=1)
    b2 = b_attn.reshape(1, 1)
    bn = b_node.reshape(1, D)
    srcr = edge_index[0].reshape(NS * NSTRIP, S)
    dstr = edge_index[1].reshape(NS * NSTRIP, S)

    am, a2 = pl.pallas_call(
        _pre_body,
        grid=(T,),
        in_specs=[
            pl.BlockSpec((1, N, D), lambda t: (t, 0, 0)),
            pl.BlockSpec((1, 1, N), lambda t: (t, 0, 0)),
            pl.BlockSpec((D, 2), lambda t: (0, 0)),
            pl.BlockSpec((1, 1), lambda t: (0, 0)),
        ],
        out_specs=[
            pl.BlockSpec((1, 1, N), lambda t: (t, 0, 0)),
            pl.BlockSpec((1, 1, N), lambda t: (t, 0, 0)),
        ],
        out_shape=[
            jax.ShapeDtypeStruct((T, 1, N), jnp.float32),
            jax.ShapeDtypeStruct((T, 1, N), jnp.float32),
        ],
    )(nfT, evT, wcat, b2)

    numer, denom = _get_sc_kernel()(nf_flat, am.reshape(T * N),
                                    a2.reshape(T * N), srcr, dstr)
    numer = numer.reshape(T, NP, D)[:, :N, :]
    denom = denom.reshape(T, NP)[:, :N]

    outT = pl.pallas_call(
        _post_body,
        grid=(T,),
        in_specs=[
            pl.BlockSpec((1, N, D), lambda t: (t, 0, 0)),
            pl.BlockSpec((1, N, D), lambda t: (t, 0, 0)),
            pl.BlockSpec((1, 1, N), lambda t: (t, 0, 0)),
            pl.BlockSpec((2 * D, D), lambda t: (0, 0)),
            pl.BlockSpec((1, D), lambda t: (0, 0)),
        ],
        out_specs=pl.BlockSpec((1, N, D), lambda t: (t, 0, 0)),
        out_shape=jax.ShapeDtypeStruct((T, N, D), jnp.float32),
    )(nfT, numer, denom.reshape(T, 1, N), W_node, bn)

    return jnp.transpose(outT, (1, 0, 2))


# speedup vs baseline: 3.0755x; 3.0755x over previous
import functools

import jax
import jax.numpy as jnp
from jax import lax
from jax.experimental import pallas as pl
from jax.experimental.pallas import tpu as pltpu
from jax.experimental.pallas import tpu_sc as plsc

N = 10000
E = 320000
D = 128
T = 4

NC = 2
NS = 16
LN = 16

EPT = E // NS
K = 80
S = 2000
NSPC = S // K
NSTRIP = EPT // S
GPR = K // LN
TPC = T // NC
NP = 10240
RPT = NP // NS


def _pre_body(nfT_ref, evT_ref, wcat_ref, b_ref, am_ref, a2_ref):
    x = nfT_ref[0]
    a = jnp.dot(x, wcat_ref[...], preferred_element_type=jnp.float32)
    ev = evT_ref[0, 0]
    a1 = a[:, 0] + b_ref[0, 0]
    am_ref[0, 0] = jnp.where(ev == 1, a1, jnp.float32(-1e9))
    a2_ref[0, 0] = a[:, 1]


def _post_body(nfT_ref, num_ref, den_ref, wn_ref, bn_ref, out_ref):
    x = nfT_ref[0]
    den = jnp.maximum(den_ref[0, 0], jnp.float32(1e-9))
    agg = num_ref[0] / den[:, None]
    h = (jnp.dot(x, wn_ref[:D, :], preferred_element_type=jnp.float32)
         + jnp.dot(agg, wn_ref[D:, :], preferred_element_type=jnp.float32)
         + bn_ref[0])
    out_ref[0] = jnp.maximum(h, 0.0)


def _sc_body(nf_flat, am_hbm, a2_hbm, src_hbm, dst_hbm,
             numer_out, denom_out,
             numer_sh, denom_sh, am_sh, a2_sh,
             src_v, dst_v, amg_v, a2g_v, w_v, rows_v, zbd_v, gsem, ssem):
    cid = lax.axis_index("c")
    sid = lax.axis_index("s")

    zv = jnp.zeros((LN,), jnp.float32)

    def _zero_zbd(j, _):
        zbd_v[pl.ds(j * LN, LN)] = zv
        return 0
    lax.fori_loop(0, RPT // LN, _zero_zbd, 0)

    for i in range(TPC):
        t = cid * TPC + i
        tbase = t * N

        def _zero_rows(j, _):
            for k in range(D // LN):
                rows_v[0, j, k * LN:(k + 1) * LN] = zv
            return 0
        lax.fori_loop(0, K, _zero_rows, 0)

        for r in range(RPT // K):
            pltpu.sync_copy(rows_v.at[0],
                            numer_sh.at[pl.ds(sid * RPT + r * K, K)])
        pltpu.sync_copy(zbd_v, denom_sh.at[pl.ds(sid * RPT, RPT)])

        @pl.when(sid == 0)
        def _():
            pltpu.sync_copy(am_hbm.at[pl.ds(t * N, N)], am_sh)

        @pl.when(sid == 1)
        def _():
            pltpu.sync_copy(a2_hbm.at[pl.ds(t * N, N)], a2_sh)

        plsc.subcore_barrier()

        def _strip(st, _):
            row = sid * NSTRIP + st
            pltpu.sync_copy(src_hbm.at[row], src_v)
            pltpu.sync_copy(dst_hbm.at[row], dst_v)

            pltpu.async_copy(am_sh.at[src_v], amg_v, gsem).wait()
            pltpu.async_copy(a2_sh.at[dst_v], a2g_v, gsem).wait()

            def _wcalc(g, _):
                sl = pl.ds(g * LN, LN)
                w_v[sl] = jnp.exp(amg_v[sl] + a2g_v[sl])
                src_v[sl] = src_v[sl] + tbase
                return 0
            lax.fori_loop(0, S // LN, _wcalc, 0)

            pltpu.sync_copy(w_v, denom_sh.at[dst_v], add=True)

            def _do_scale(k, buf):
                def _scale(g, _):
                    wg = w_v[pl.ds(k * K + g * LN, LN)]
                    for l in range(LN):
                        wj = jnp.full((LN,), wg[l])
                        j = g * LN + l
                        for c8 in range(D // LN):
                            sl = pl.ds(c8 * LN, LN)
                            rows_v[buf, j, sl] = rows_v[buf, j, sl] * wj
                    return 0
                lax.fori_loop(0, GPR, _scale, 0)

            def _gather(k, buf):
                pltpu.async_copy(
                    nf_flat.at[src_v.at[pl.ds(k * K, K)]], rows_v.at[buf],
                    gsem)

            def _gwait(k, buf):
                pltpu.make_async_copy(
                    nf_flat.at[src_v.at[pl.ds(k * K, K)]], rows_v.at[buf],
                    gsem).wait()

            def _scatter(k, buf):
                pltpu.async_copy(rows_v.at[buf],
                                 numer_sh.at[dst_v.at[pl.ds(k * K, K)]],
                                 ssem, add=True)

            def _swait(k, buf):
                pltpu.make_async_copy(
                    rows_v.at[buf],
                    numer_sh.at[dst_v.at[pl.ds(k * K, K)]], ssem).wait()

            _gather(0, 0)
            _gwait(0, 0)
            _gather(1, 1)
            _do_scale(0, 0)
            _scatter(0, 0)
            _gwait(1, 1)
            _gather(2, 2)
            _do_scale(1, 1)
            _scatter(1, 1)
            _gwait(2, 2)
            _swait(0, 0)
            _gather(3, 0)
            _do_scale(2, 2)
            _scatter(2, 2)

            def _triple(q, _):
                k0 = 3 * q
                for j in range(3):
                    k = k0 + j
                    b = j
                    bn = (j + 1) % 3
                    _gwait(k, b)
                    _swait(k - 2, bn)
                    _gather(k + 1, bn)
                    _do_scale(k, b)
                    _scatter(k, b)
                return 0
            lax.fori_loop(1, (NSPC - 1) // 3, _triple, 0)

            kt = NSPC - 1
            _gwait(kt, 0)
            _swait(kt - 2, 1)
            _do_scale(kt, 0)
            _scatter(kt, 0)
            _swait(kt - 1, 2)
            _swait(kt, 0)
            return 0
        lax.fori_loop(0, NSTRIP, _strip, 0)

        plsc.subcore_barrier()

        pbase = t * NP
        pltpu.sync_copy(numer_sh.at[pl.ds(sid * RPT, RPT)],
                        numer_out.at[pl.ds(pbase + sid * RPT, RPT)])
        pltpu.sync_copy(denom_sh.at[pl.ds(sid * RPT, RPT)],
                        denom_out.at[pl.ds(pbase + sid * RPT, RPT)])

        plsc.subcore_barrier()


@functools.cache
def _get_sc_kernel():
    @functools.partial(
        pl.kernel,
        out_type=(jax.ShapeDtypeStruct((T * NP, D), jnp.float32),
                  jax.ShapeDtypeStruct((T * NP,), jnp.float32)),
        mesh=plsc.VectorSubcoreMesh(core_axis_name="c", subcore_axis_name="s",
                                    num_cores=NC, num_subcores=NS),
        scratch_types=(
            pltpu.VMEM_SHARED((NP, D), jnp.float32),
            pltpu.VMEM_SHARED((NP,), jnp.float32),
            pltpu.VMEM_SHARED((N,), jnp.float32),
            pltpu.VMEM_SHARED((N,), jnp.float32),
            pltpu.VMEM((S,), jnp.int32),
            pltpu.VMEM((S,), jnp.int32),
            pltpu.VMEM((S,), jnp.float32),
            pltpu.VMEM((S,), jnp.float32),
            pltpu.VMEM((S,), jnp.float32),
            pltpu.VMEM((3, K, D), jnp.float32),
            pltpu.VMEM((RPT,), jnp.float32),
            pltpu.SemaphoreType.DMA,
            pltpu.SemaphoreType.DMA,
        ),
        compiler_params=pltpu.CompilerParams(needs_layout_passes=False,
                                             use_tc_tiling_on_sc=False),
    )
    def _sc_kernel(nf_flat, am, a2, srcr, dstr, numer_out, denom_out, *scratch):
        _sc_body(nf_flat, am, a2, srcr, dstr, numer_out, denom_out, *scratch)

    return _sc_kernel


def kernel(nf, edge_index, input_events, W_attn, b_attn, W_node, b_node):
    nfT = jnp.transpose(nf, (1, 0, 2))
    nf_flat = nfT.reshape(T * N, D)
    evT = jnp.transpose(input_events, (1, 0)).reshape(T, 1, N)
    wcat = jnp.concatenate([W_attn[:D], W_attn[D:]], axis=1)
    b2 = b_attn.reshape(1, 1)
    bn = b_node.reshape(1, D)
    srcr = edge_index[0].reshape(NS * NSTRIP, S)
    dstr = edge_index[1].reshape(NS * NSTRIP, S)

    am, a2 = pl.pallas_call(
        _pre_body,
        grid=(T,),
        in_specs=[
            pl.BlockSpec((1, N, D), lambda t: (t, 0, 0)),
            pl.BlockSpec((1, 1, N), lambda t: (t, 0, 0)),
            pl.BlockSpec((D, 2), lambda t: (0, 0)),
            pl.BlockSpec((1, 1), lambda t: (0, 0)),
        ],
        out_specs=[
            pl.BlockSpec((1, 1, N), lambda t: (t, 0, 0)),
            pl.BlockSpec((1, 1, N), lambda t: (t, 0, 0)),
        ],
        out_shape=[
            jax.ShapeDtypeStruct((T, 1, N), jnp.float32),
            jax.ShapeDtypeStruct((T, 1, N), jnp.float32),
        ],
    )(nfT, evT, wcat, b2)

    numer, denom = _get_sc_kernel()(nf_flat, am.reshape(T * N),
                                    a2.reshape(T * N), srcr, dstr)
    numer = numer.reshape(T, NP, D)[:, :N, :]
    denom = denom.reshape(T, NP)[:, :N]

    outT = pl.pallas_call(
        _post_body,
        grid=(T,),
        in_specs=[
            pl.BlockSpec((1, N, D), lambda t: (t, 0, 0)),
            pl.BlockSpec((1, N, D), lambda t: (t, 0, 0)),
            pl.BlockSpec((1, 1, N), lambda t: (t, 0, 0)),
            pl.BlockSpec((2 * D, D), lambda t: (0, 0)),
            pl.BlockSpec((1, D), lambda t: (0, 0)),
        ],
        out_specs=pl.BlockSpec((1, N, D), lambda t: (t, 0, 0)),
        out_shape=jax.ShapeDtypeStruct((T, N, D), jnp.float32),
    )(nfT, numer, denom.reshape(T, 1, N), W_node, bn)

    return jnp.transpose(outT, (1, 0, 2))


# speedup vs baseline: 3.1709x; 1.0310x over previous
import functools

import jax
import jax.numpy as jnp
from jax import lax
from jax.experimental import pallas as pl
from jax.experimental.pallas import tpu as pltpu
from jax.experimental.pallas import tpu_sc as plsc

N = 10000
E = 320000
D = 128
T = 4

NC = 2
NS = 16
LN = 16

EPT = E // NS
K = 80
S = 2000
NSPC = S // K
NSTRIP = EPT // S
GPR = K // LN
TPC = T // NC
NP = 10240
RPT = NP // NS


def _pre_body(nfT_ref, evT_ref, wcat_ref, b_ref, am_ref, a2_ref):
    x = nfT_ref[0]
    a = jnp.dot(x, wcat_ref[...], preferred_element_type=jnp.float32)
    ev = evT_ref[0, 0]
    a1 = a[:, 0] + b_ref[0, 0]
    am_ref[0, 0] = jnp.where(ev == 1, a1, jnp.float32(-1e9))
    a2_ref[0, 0] = a[:, 1]


def _post_body(nfT_ref, num_ref, den_ref, wn_ref, bn_ref, out_ref):
    x = nfT_ref[0]
    den = jnp.maximum(den_ref[0, 0], jnp.float32(1e-9))
    agg = num_ref[0] / den[:, None]
    h = (jnp.dot(x, wn_ref[:D, :], preferred_element_type=jnp.float32)
         + jnp.dot(agg, wn_ref[D:, :], preferred_element_type=jnp.float32)
         + bn_ref[0])
    out_ref[0] = jnp.maximum(h, 0.0)


def _sc_body(nf_flat, am_hbm, a2_hbm, src_hbm, dst_hbm,
             numer_out, denom_out,
             numer_sh, denom_sh, am_sh, a2_sh,
             src_v, dst_v, amg_v, a2g_v, w_v, rows_v, zbd_v, gsem, ssem,
             dsem):
    cid = lax.axis_index("c")
    sid = lax.axis_index("s")

    zv = jnp.zeros((LN,), jnp.float32)

    def _zero_zbd(j, _):
        zbd_v[pl.ds(j * LN, LN)] = zv
        return 0
    lax.fori_loop(0, RPT // LN, _zero_zbd, 0)

    for i in range(TPC):
        t = cid * TPC + i
        tbase = t * N

        def _zero_rows(j, _):
            for k in range(D // LN):
                rows_v[0, j, k * LN:(k + 1) * LN] = zv
            return 0
        lax.fori_loop(0, K, _zero_rows, 0)

        for r in range(RPT // K):
            pltpu.sync_copy(rows_v.at[0],
                            numer_sh.at[pl.ds(sid * RPT + r * K, K)])
        pltpu.sync_copy(zbd_v, denom_sh.at[pl.ds(sid * RPT, RPT)])

        @pl.when(sid == 0)
        def _():
            pltpu.sync_copy(am_hbm.at[pl.ds(t * N, N)], am_sh)

        @pl.when(sid == 1)
        def _():
            pltpu.sync_copy(a2_hbm.at[pl.ds(t * N, N)], a2_sh)

        plsc.subcore_barrier()

        def _strip(st, _):
            row = sid * NSTRIP + st
            pltpu.async_copy(src_hbm.at[row], src_v, gsem)
            pltpu.async_copy(dst_hbm.at[row], dst_v, gsem)
            pltpu.make_async_copy(src_hbm.at[row], src_v, gsem).wait()
            pltpu.make_async_copy(dst_hbm.at[row], dst_v, gsem).wait()

            pltpu.async_copy(am_sh.at[src_v], amg_v, gsem)
            pltpu.async_copy(a2_sh.at[dst_v], a2g_v, gsem)
            pltpu.make_async_copy(am_sh.at[src_v], amg_v, gsem).wait()
            pltpu.make_async_copy(a2_sh.at[dst_v], a2g_v, gsem).wait()

            def _wcalc(g, _):
                sl = pl.ds(g * LN, LN)
                w_v[sl] = jnp.exp(amg_v[sl] + a2g_v[sl])
                src_v[sl] = src_v[sl] + tbase
                return 0
            lax.fori_loop(0, S // LN, _wcalc, 0)

            pltpu.async_copy(w_v, denom_sh.at[dst_v], dsem, add=True)

            def _do_scale(k, buf):
                def _scale(g, _):
                    wg = w_v[pl.ds(k * K + g * LN, LN)]
                    for l in range(LN):
                        wj = jnp.full((LN,), wg[l])
                        j = g * LN + l
                        for c8 in range(D // LN):
                            sl = pl.ds(c8 * LN, LN)
                            rows_v[buf, j, sl] = rows_v[buf, j, sl] * wj
                    return 0
                lax.fori_loop(0, GPR, _scale, 0)

            def _gather(k, buf):
                pltpu.async_copy(
                    nf_flat.at[src_v.at[pl.ds(k * K, K)]], rows_v.at[buf],
                    gsem)

            def _gwait(k, buf):
                pltpu.make_async_copy(
                    nf_flat.at[src_v.at[pl.ds(k * K, K)]], rows_v.at[buf],
                    gsem).wait()

            def _scatter(k, buf):
                pltpu.async_copy(rows_v.at[buf],
                                 numer_sh.at[dst_v.at[pl.ds(k * K, K)]],
                                 ssem, add=True)

            def _swait(k, buf):
                pltpu.make_async_copy(
                    rows_v.at[buf],
                    numer_sh.at[dst_v.at[pl.ds(k * K, K)]], ssem).wait()

            _gather(0, 0)
            _gwait(0, 0)
            _gather(1, 1)
            _do_scale(0, 0)
            _scatter(0, 0)
            _gwait(1, 1)
            _gather(2, 2)
            _do_scale(1, 1)
            _scatter(1, 1)
            _gwait(2, 2)
            _swait(0, 0)
            _gather(3, 0)
            _do_scale(2, 2)
            _scatter(2, 2)

            def _triple(q, _):
                k0 = 3 * q
                for j in range(3):
                    k = k0 + j
                    b = j
                    bn = (j + 1) % 3
                    _gwait(k, b)
                    _swait(k - 2, bn)
                    _gather(k + 1, bn)
                    _do_scale(k, b)
                    _scatter(k, b)
                return 0
            lax.fori_loop(1, (NSPC - 1) // 3, _triple, 0)

            kt = NSPC - 1
            _gwait(kt, 0)
            _swait(kt - 2, 1)
            _do_scale(kt, 0)
            _scatter(kt, 0)
            _swait(kt - 1, 2)
            _swait(kt, 0)
            pltpu.make_async_copy(w_v, denom_sh.at[dst_v], dsem).wait()
            return 0
        lax.fori_loop(0, NSTRIP, _strip, 0)

        plsc.subcore_barrier()

        pbase = t * NP
        pltpu.sync_copy(numer_sh.at[pl.ds(sid * RPT, RPT)],
                        numer_out.at[pl.ds(pbase + sid * RPT, RPT)])
        pltpu.sync_copy(denom_sh.at[pl.ds(sid * RPT, RPT)],
                        denom_out.at[pl.ds(pbase + sid * RPT, RPT)])

        plsc.subcore_barrier()


@functools.cache
def _get_sc_kernel():
    @functools.partial(
        pl.kernel,
        out_type=(jax.ShapeDtypeStruct((T * NP, D), jnp.float32),
                  jax.ShapeDtypeStruct((T * NP,), jnp.float32)),
        mesh=plsc.VectorSubcoreMesh(core_axis_name="c", subcore_axis_name="s",
                                    num_cores=NC, num_subcores=NS),
        scratch_types=(
            pltpu.VMEM_SHARED((NP, D), jnp.float32),
            pltpu.VMEM_SHARED((NP,), jnp.float32),
            pltpu.VMEM_SHARED((N,), jnp.float32),
            pltpu.VMEM_SHARED((N,), jnp.float32),
            pltpu.VMEM((S,), jnp.int32),
            pltpu.VMEM((S,), jnp.int32),
            pltpu.VMEM((S,), jnp.float32),
            pltpu.VMEM((S,), jnp.float32),
            pltpu.VMEM((S,), jnp.float32),
            pltpu.VMEM((3, K, D), jnp.float32),
            pltpu.VMEM((RPT,), jnp.float32),
            pltpu.SemaphoreType.DMA,
            pltpu.SemaphoreType.DMA,
            pltpu.SemaphoreType.DMA,
        ),
        compiler_params=pltpu.CompilerParams(needs_layout_passes=False,
                                             use_tc_tiling_on_sc=False),
    )
    def _sc_kernel(nf_flat, am, a2, srcr, dstr, numer_out, denom_out, *scratch):
        _sc_body(nf_flat, am, a2, srcr, dstr, numer_out, denom_out, *scratch)

    return _sc_kernel


def kernel(nf, edge_index, input_events, W_attn, b_attn, W_node, b_node):
    nfT = jnp.transpose(nf, (1, 0, 2))
    nf_flat = nfT.reshape(T * N, D)
    evT = jnp.transpose(input_events, (1, 0)).reshape(T, 1, N)
    wcat = jnp.concatenate([W_attn[:D], W_attn[D:]], axis=1)
    b2 = b_attn.reshape(1, 1)
    bn = b_node.reshape(1, D)
    srcr = edge_index[0].reshape(NS * NSTRIP, S)
    dstr = edge_index[1].reshape(NS * NSTRIP, S)

    am, a2 = pl.pallas_call(
        _pre_body,
        grid=(T,),
        in_specs=[
            pl.BlockSpec((1, N, D), lambda t: (t, 0, 0)),
            pl.BlockSpec((1, 1, N), lambda t: (t, 0, 0)),
            pl.BlockSpec((D, 2), lambda t: (0, 0)),
            pl.BlockSpec((1, 1), lambda t: (0, 0)),
        ],
        out_specs=[
            pl.BlockSpec((1, 1, N), lambda t: (t, 0, 0)),
            pl.BlockSpec((1, 1, N), lambda t: (t, 0, 0)),
        ],
        out_shape=[
            jax.ShapeDtypeStruct((T, 1, N), jnp.float32),
            jax.ShapeDtypeStruct((T, 1, N), jnp.float32),
        ],
    )(nfT, evT, wcat, b2)

    numer, denom = _get_sc_kernel()(nf_flat, am.reshape(T * N),
                                    a2.reshape(T * N), srcr, dstr)
    numer = numer.reshape(T, NP, D)[:, :N, :]
    denom = denom.reshape(T, NP)[:, :N]

    outT = pl.pallas_call(
        _post_body,
        grid=(T,),
        in_specs=[
            pl.BlockSpec((1, N, D), lambda t: (t, 0, 0)),
            pl.BlockSpec((1, N, D), lambda t: (t, 0, 0)),
            pl.BlockSpec((1, 1, N), lambda t: (t, 0, 0)),
            pl.BlockSpec((2 * D, D), lambda t: (0, 0)),
            pl.BlockSpec((1, D), lambda t: (0, 0)),
        ],
        out_specs=pl.BlockSpec((1, N, D), lambda t: (t, 0, 0)),
        out_shape=jax.ShapeDtypeStruct((T, N, D), jnp.float32),
    )(nfT, numer, denom.reshape(T, 1, N), W_node, bn)

    return jnp.transpose(outT, (1, 0, 2))
